# Initial kernel scaffold; baseline (speedup 1.0000x reference)
import jax
import jax.numpy as jnp
from jax.experimental import pallas as pl


def kernel(node_feature, edge_index, ef_init, W, b):
    raise NotImplementedError("write your pallas kernel here")


# speedup vs baseline: 6.9441x; 6.9441x over previous
import jax
import jax.numpy as jnp
from jax import lax
from jax.experimental import pallas as pl
from jax.experimental.pallas import tpu as pltpu
from jax.experimental.pallas import tpu_sc as plsc

N_NODES = 10000
N_EDGES = 160000
D_FEAT = 256
D_EDGE = 16

LANES = 16
NW = 16
E_W = N_EDGES // NW
GROUPS = E_W // LANES
ROWS = N_EDGES // LANES


def _tc_scores_body(nf_ref, wsd_ref, ef2_ref, wdiag_ref, b_ref, s2_ref, e3_ref):
    s2_ref[...] = jnp.dot(nf_ref[...], wsd_ref[...],
                          preferred_element_type=jnp.float32,
                          precision=lax.Precision.HIGHEST)
    e3_ref[...] = jnp.dot(ef2_ref[...], wdiag_ref[...],
                          preferred_element_type=jnp.float32,
                          precision=lax.Precision.HIGHEST) + b_ref[0]


def _sc_softmax_body(s2_hbm, i0_hbm, i1_hbm, e_hbm, out_hbm,
                     table_v, i0_v, i1_v, e_v, l_v, red_v, stage_v,
                     shared_max, shared_sum):
    wid = lax.axis_index("s")
    base = wid * E_W

    pltpu.sync_copy(s2_hbm, table_v)
    pltpu.sync_copy(i0_hbm.at[pl.ds(base, E_W)], i0_v)
    pltpu.sync_copy(i1_hbm.at[pl.ds(base, E_W)], i1_v)
    pltpu.sync_copy(e_hbm.at[pl.ds(base, E_W)], e_v)

    def pass1(j, m):
        ds = pl.ds(j * LANES, LANES)
        s0 = plsc.load_gather(table_v, [i0_v[ds] * 2])
        s1 = plsc.load_gather(table_v, [i1_v[ds] * 2 + 1])
        logit = jnp.maximum(s0 + s1 + e_v[ds], 0.0)
        l_v[ds] = logit
        return jnp.maximum(m, logit)

    m = lax.fori_loop(0, GROUPS, pass1, jnp.zeros((LANES,), jnp.float32))
    stage_v[...] = m
    pltpu.sync_copy(stage_v, shared_max.at[pl.ds(wid * LANES, LANES)])
    plsc.subcore_barrier()
    pltpu.sync_copy(shared_max, red_v)
    g = red_v[pl.ds(0, LANES)]
    for i in range(1, NW):
        g = jnp.maximum(g, red_v[pl.ds(i * LANES, LANES)])
    gmax = jnp.max(g)

    def pass2(j, s):
        ds = pl.ds(j * LANES, LANES)
        t = jnp.exp(l_v[ds] - gmax)
        l_v[ds] = t
        return s + t

    s = lax.fori_loop(0, GROUPS, pass2, jnp.zeros((LANES,), jnp.float32))
    stage_v[...] = s
    pltpu.sync_copy(stage_v, shared_sum.at[pl.ds(wid * LANES, LANES)])
    plsc.subcore_barrier()
    pltpu.sync_copy(shared_sum, red_v)
    a = red_v[pl.ds(0, LANES)]
    for i in range(1, NW):
        a = a + red_v[pl.ds(i * LANES, LANES)]
    inv = jnp.ones((LANES,), jnp.float32) / jnp.broadcast_to(jnp.sum(a), (LANES,))

    def pass3(j, _):
        ds = pl.ds(j * LANES, LANES)
        l_v[ds] = l_v[ds] * inv
        return 0

    lax.fori_loop(0, GROUPS, pass3, 0)
    pltpu.sync_copy(l_v, out_hbm.at[pl.ds(base, E_W)])


def kernel(node_feature, edge_index, ef_init, W, b):
    idx = edge_index.astype(jnp.int32)
    i0 = idx[0]
    i1 = idx[1]
    w = W[:, 0]
    wsd = jnp.stack([w[:D_FEAT], w[D_FEAT:2 * D_FEAT]], axis=1)
    we = w[2 * D_FEAT:]
    eye = jnp.eye(LANES, dtype=jnp.float32)
    wdiag = (eye[:, None, :] * we[None, :, None]).reshape(D_EDGE * LANES, LANES)
    ef2 = ef_init.reshape(ROWS, D_EDGE * LANES)

    blk = 2000
    s2, e3 = pl.pallas_call(
        _tc_scores_body,
        grid=(N_NODES // blk,),
        out_shape=(jax.ShapeDtypeStruct((N_NODES, 2), jnp.float32),
                   jax.ShapeDtypeStruct((ROWS, LANES), jnp.float32)),
        in_specs=[pl.BlockSpec((blk, D_FEAT), lambda i: (i, 0)),
                  pl.BlockSpec((D_FEAT, 2), lambda i: (0, 0)),
                  pl.BlockSpec((blk, D_EDGE * LANES), lambda i: (i, 0)),
                  pl.BlockSpec((D_EDGE * LANES, LANES), lambda i: (0, 0)),
                  pl.BlockSpec(memory_space=pltpu.SMEM)],
        out_specs=(pl.BlockSpec((blk, 2), lambda i: (i, 0)),
                   pl.BlockSpec((blk, LANES), lambda i: (i, 0))),
    )(node_feature, wsd, ef2, wdiag, b)
    e = e3.reshape(N_EDGES)
    table = s2.reshape(2 * N_NODES)

    mesh = plsc.VectorSubcoreMesh(core_axis_name="c", subcore_axis_name="s",
                                  num_cores=1)
    out = pl.kernel(
        _sc_softmax_body,
        out_type=jax.ShapeDtypeStruct((N_EDGES,), jnp.float32),
        mesh=mesh,
        compiler_params=pltpu.CompilerParams(needs_layout_passes=False),
        scratch_types=[
            pltpu.VMEM((2 * N_NODES,), jnp.float32),
            pltpu.VMEM((E_W,), jnp.int32),
            pltpu.VMEM((E_W,), jnp.int32),
            pltpu.VMEM((E_W,), jnp.float32),
            pltpu.VMEM((E_W,), jnp.float32),
            pltpu.VMEM((NW * LANES,), jnp.float32),
            pltpu.VMEM((LANES,), jnp.float32),
            pltpu.VMEM_SHARED((NW * LANES,), jnp.float32),
            pltpu.VMEM_SHARED((NW * LANES,), jnp.float32),
        ],
    )(table, i0, i1, e)
    return out
